# Initial kernel scaffold; baseline (speedup 1.0000x reference)
import jax
import jax.numpy as jnp
from jax.experimental import pallas as pl


def kernel(x, edge_index, batch, W1, b1, W2, b2, Wh, bh):
    raise NotImplementedError("write your pallas kernel here")


# speedup vs baseline: 5.4979x; 5.4979x over previous
import functools

import jax
import jax.numpy as jnp
from jax import lax
from jax.experimental import pallas as pl
from jax.experimental.pallas import tpu as pltpu
from jax.experimental.pallas import tpu_sc as plsc

N = 10000
E = 320000
D = 128
H = 128
G = 64

NC = 2
NS = 16
NW = NC * NS

EPT = E // NW
K = 80
NCHUNK = EPT // K
NP = 10240
RPT = NP // NS
RCH = 128
NRCH = RPT // RCH


def _sc_body(x_hbm, src_hbm, dst_hbm, out_hbm, src_v, dst_v, rows_v, zbuf,
             agg_sh, sem):
    cid = lax.axis_index("c")
    sid = lax.axis_index("s")

    def _zfill(i, carry):
        for j in range(8):
            zbuf[i, pl.ds(j * 16, 16)] = jnp.zeros((16,), jnp.float32)
        return carry

    lax.fori_loop(0, RCH, _zfill, 0)
    for r in range(NRCH):
        pltpu.sync_copy(zbuf, agg_sh.at[pl.ds(sid * RPT + r * RCH, RCH)])
    plsc.subcore_barrier()

    ebase = (cid * NS + sid) * EPT

    def _edge_chunk(c, carry):
        base = ebase + c * K
        pltpu.sync_copy(src_hbm.at[pl.ds(base, K)], src_v)
        pltpu.sync_copy(dst_hbm.at[pl.ds(base, K)], dst_v)
        pltpu.async_copy(x_hbm.at[src_v], rows_v, sem).wait()
        pltpu.sync_copy(rows_v, agg_sh.at[dst_v], add=True)
        return carry

    lax.fori_loop(0, NCHUNK, _edge_chunk, 0)
    plsc.subcore_barrier()

    for r in range(NRCH):
        row0 = sid * RPT + r * RCH
        pltpu.sync_copy(agg_sh.at[pl.ds(row0, RCH)], zbuf)
        pltpu.sync_copy(zbuf, out_hbm.at[cid, pl.ds(row0, RCH)])


_sc_scatter = functools.partial(
    pl.kernel,
    out_type=jax.ShapeDtypeStruct((NC, NP, D), jnp.float32),
    mesh=plsc.VectorSubcoreMesh(
        core_axis_name="c", subcore_axis_name="s", num_cores=NC, num_subcores=NS
    ),
    scratch_types=[
        pltpu.VMEM((K,), jnp.int32),
        pltpu.VMEM((K,), jnp.int32),
        pltpu.VMEM((K, D), jnp.float32),
        pltpu.VMEM((RCH, D), jnp.float32),
        pltpu.VMEM_SHARED((NP, D), jnp.float32),
        pltpu.SemaphoreType.DMA,
    ],
)(_sc_body)


BLK = 1000
NBLK = N // BLK


def _tc_body(x_ref, p_ref, b_ref, W1_ref, b1_ref, W2_ref, b2_ref, Wh_ref,
             bh_ref, nl_ref, gl_ref):
    i = pl.program_id(0)
    h0 = x_ref[...] + p_ref[0] + p_ref[1]
    h1 = jnp.dot(h0, W1_ref[...], preferred_element_type=jnp.float32) + b1_ref[...]
    h1 = jnp.maximum(h1, 0.0)
    h2 = jnp.dot(h1, W2_ref[...], preferred_element_type=jnp.float32) + b2_ref[...]
    nl = jnp.dot(h2, Wh_ref[...], preferred_element_type=jnp.float32) + bh_ref[...]
    nl_ref[...] = nl

    seg = b_ref[0, 0, :]
    gids = lax.broadcasted_iota(jnp.int32, (G, BLK), 0)
    onehot = (gids == seg[None, :]).astype(jnp.float32)
    part = jnp.dot(onehot, nl, preferred_element_type=jnp.float32)

    @pl.when(i == 0)
    def _():
        gl_ref[...] = jnp.zeros_like(gl_ref)

    gl_ref[...] += part


def _tc_mlp(x, parts, batch3, W1, b1r, W2, b2r, Wh, bhr):
    full = lambda shape: pl.BlockSpec(shape, lambda i: tuple(0 for _ in shape))
    return pl.pallas_call(
        _tc_body,
        grid=(NBLK,),
        in_specs=[
            pl.BlockSpec((BLK, D), lambda i: (i, 0)),
            pl.BlockSpec((NC, BLK, D), lambda i: (0, i, 0)),
            pl.BlockSpec((1, 1, BLK), lambda i: (i, 0, 0)),
            full((D, H)),
            full((1, H)),
            full((H, H)),
            full((1, H)),
            full((H, 1)),
            full((1, 1)),
        ],
        out_specs=[
            pl.BlockSpec((BLK, 1), lambda i: (i, 0)),
            pl.BlockSpec((G, 1), lambda i: (0, 0)),
        ],
        out_shape=[
            jax.ShapeDtypeStruct((N, 1), jnp.float32),
            jax.ShapeDtypeStruct((G, 1), jnp.float32),
        ],
    )(x, parts, batch3, W1, b1r, W2, b2r, Wh, bhr)


def kernel(x, edge_index, batch, W1, b1, W2, b2, Wh, bh):
    parts = _sc_scatter(x, edge_index[0], edge_index[1])
    batch3 = batch.reshape(NBLK, 1, BLK)
    nl, gl = _tc_mlp(
        x, parts, batch3, W1, b1.reshape(1, H), W2, b2.reshape(1, H), Wh,
        bh.reshape(1, 1),
    )
    return (gl, nl)


# speedup vs baseline: 8.5344x; 1.5523x over previous
import functools

import jax
import jax.numpy as jnp
from jax import lax
from jax.experimental import pallas as pl
from jax.experimental.pallas import tpu as pltpu
from jax.experimental.pallas import tpu_sc as plsc

N = 10000
E = 320000
D = 128
H = 128
G = 64

NC = 2
NS = 16
NW = NC * NS

EPT = E // NW
K = 80
NCHUNK = EPT // K
NP = 10240
RPT = NP // NS
RCH = 128
NRCH = RPT // RCH


def _sc_body(x_hbm, src_hbm, dst_hbm, out_hbm, src_v, dst_v, rows_v, zbuf,
             agg_sh, gsems):
    cid = lax.axis_index("c")
    sid = lax.axis_index("s")

    def _zfill(i, carry):
        for j in range(8):
            zbuf[i, pl.ds(j * 16, 16)] = jnp.zeros((16,), jnp.float32)
        return carry

    lax.fori_loop(0, RCH, _zfill, 0)
    for r in range(NRCH):
        pltpu.sync_copy(zbuf, agg_sh.at[pl.ds(sid * RPT + r * RCH, RCH)])
    plsc.subcore_barrier()

    ebase = (cid * NS + sid) * EPT

    def _load_idx(c, b):
        pltpu.sync_copy(src_hbm.at[pl.ds(ebase + c * K, K)], src_v.at[b])
        pltpu.sync_copy(dst_hbm.at[pl.ds(ebase + c * K, K)], dst_v.at[b])

    def _g_issue(b):
        pltpu.async_copy(x_hbm.at[src_v.at[b]], rows_v.at[b], gsems[b])

    def _g_wait(b):
        pltpu.make_async_copy(x_hbm.at[src_v.at[b]], rows_v.at[b],
                              gsems[b]).wait()

    def _scatter(b):
        pltpu.sync_copy(rows_v.at[b], agg_sh.at[dst_v.at[b]], add=True)

    _load_idx(0, 0)
    _g_issue(0)

    def _steady(g, carry):
        for u in range(2):
            c = 2 * g + u
            b = u
            _load_idx(c + 1, 1 - b)
            _g_issue(1 - b)
            _g_wait(b)
            _scatter(b)
        return carry

    lax.fori_loop(0, (NCHUNK - 1) // 2, _steady, 0)
    _g_wait(0)
    _scatter(0)
    plsc.subcore_barrier()

    for r in range(NRCH):
        row0 = sid * RPT + r * RCH
        pltpu.sync_copy(agg_sh.at[pl.ds(row0, RCH)], zbuf)
        pltpu.sync_copy(zbuf, out_hbm.at[cid, pl.ds(row0, RCH)])


_sc_scatter = functools.partial(
    pl.kernel,
    out_type=jax.ShapeDtypeStruct((NC, NP, D), jnp.float32),
    mesh=plsc.VectorSubcoreMesh(
        core_axis_name="c", subcore_axis_name="s", num_cores=NC, num_subcores=NS
    ),
    scratch_types=[
        pltpu.VMEM((2, K), jnp.int32),
        pltpu.VMEM((2, K), jnp.int32),
        pltpu.VMEM((2, K, D), jnp.float32),
        pltpu.VMEM((RCH, D), jnp.float32),
        pltpu.VMEM_SHARED((NP, D), jnp.float32),
        [pltpu.SemaphoreType.DMA] * 2,
    ],
)(_sc_body)


BLK = 1000
NBLK = N // BLK


def _tc_body(x_ref, p_ref, b_ref, W1_ref, b1_ref, W2_ref, b2_ref, Wh_ref,
             bh_ref, nl_ref, gl_ref):
    i = pl.program_id(0)
    h0 = x_ref[...] + p_ref[0] + p_ref[1]
    h1 = jnp.dot(h0, W1_ref[...], preferred_element_type=jnp.float32) + b1_ref[...]
    h1 = jnp.maximum(h1, 0.0)
    h2 = jnp.dot(h1, W2_ref[...], preferred_element_type=jnp.float32) + b2_ref[...]
    nl = jnp.dot(h2, Wh_ref[...], preferred_element_type=jnp.float32) + bh_ref[...]
    nl_ref[...] = nl

    seg = b_ref[0, 0, :]
    gids = lax.broadcasted_iota(jnp.int32, (G, BLK), 0)
    onehot = (gids == seg[None, :]).astype(jnp.float32)
    part = jnp.dot(onehot, nl, preferred_element_type=jnp.float32)

    @pl.when(i == 0)
    def _():
        gl_ref[...] = jnp.zeros_like(gl_ref)

    gl_ref[...] += part


def _tc_mlp(x, parts, batch3, W1, b1r, W2, b2r, Wh, bhr):
    full = lambda shape: pl.BlockSpec(shape, lambda i: tuple(0 for _ in shape))
    return pl.pallas_call(
        _tc_body,
        grid=(NBLK,),
        in_specs=[
            pl.BlockSpec((BLK, D), lambda i: (i, 0)),
            pl.BlockSpec((NC, BLK, D), lambda i: (0, i, 0)),
            pl.BlockSpec((1, 1, BLK), lambda i: (i, 0, 0)),
            full((D, H)),
            full((1, H)),
            full((H, H)),
            full((1, H)),
            full((H, 1)),
            full((1, 1)),
        ],
        out_specs=[
            pl.BlockSpec((BLK, 1), lambda i: (i, 0)),
            pl.BlockSpec((G, 1), lambda i: (0, 0)),
        ],
        out_shape=[
            jax.ShapeDtypeStruct((N, 1), jnp.float32),
            jax.ShapeDtypeStruct((G, 1), jnp.float32),
        ],
    )(x, parts, batch3, W1, b1r, W2, b2r, Wh, bhr)


def kernel(x, edge_index, batch, W1, b1, W2, b2, Wh, bh):
    parts = _sc_scatter(x, edge_index[0], edge_index[1])
    batch3 = batch.reshape(NBLK, 1, BLK)
    nl, gl = _tc_mlp(
        x, parts, batch3, W1, b1.reshape(1, H), W2, b2.reshape(1, H), Wh,
        bh.reshape(1, 1),
    )
    return (gl, nl)


# speedup vs baseline: 11.9332x; 1.3982x over previous
import functools

import jax
import jax.numpy as jnp
from jax import lax
from jax.experimental import pallas as pl
from jax.experimental.pallas import tpu as pltpu
from jax.experimental.pallas import tpu_sc as plsc

N = 10000
E = 320000
D = 128
H = 128
G = 64

NC = 2
NS = 16
NW = NC * NS

EPT = E // NW
K = 80
NCHUNK = EPT // K
NP = 10240
RPT = NP // NS
RCH = 128
NRCH = RPT // RCH


def _sc_body(x_hbm, src_hbm, dst_hbm, out_hbm, src_v, dst_v, rows_v, zbuf,
             agg_sh, gsems, isems, osem):
    cid = lax.axis_index("c")
    sid = lax.axis_index("s")

    def _zfill(i, carry):
        for j in range(8):
            zbuf[i, pl.ds(j * 16, 16)] = jnp.zeros((16,), jnp.float32)
        return carry

    lax.fori_loop(0, RCH, _zfill, 0)
    for r in range(NRCH):
        pltpu.sync_copy(zbuf, agg_sh.at[pl.ds(sid * RPT + r * RCH, RCH)])
    plsc.subcore_barrier()

    ebase = (cid * NS + sid) * EPT

    def _i_issue(c, s):
        pltpu.async_copy(src_hbm.at[pl.ds(ebase + c * K, K)], src_v.at[s],
                         isems[s])
        pltpu.async_copy(dst_hbm.at[pl.ds(ebase + c * K, K)], dst_v.at[s],
                         isems[s])

    def _i_wait(c, s):
        pltpu.make_async_copy(src_hbm.at[pl.ds(ebase + c * K, K)],
                              src_v.at[s], isems[s]).wait()
        pltpu.make_async_copy(dst_hbm.at[pl.ds(ebase + c * K, K)],
                              dst_v.at[s], isems[s]).wait()

    def _g_issue(s, b):
        pltpu.async_copy(x_hbm.at[src_v.at[s]], rows_v.at[b], gsems[b])

    def _g_wait(s, b):
        pltpu.make_async_copy(x_hbm.at[src_v.at[s]], rows_v.at[b],
                              gsems[b]).wait()

    def _scatter(s, b):
        pltpu.sync_copy(rows_v.at[b], agg_sh.at[dst_v.at[s]], add=True)

    def _step(c, b, s, do_next, do_pref):
        if do_next:
            _i_wait(c + 1, (s + 1) % 3)
            _g_issue((s + 1) % 3, 1 - b)
        _g_wait(s, b)
        _scatter(s, b)
        if do_pref:
            _i_issue(c + 3, s)

    _i_issue(0, 0)
    _i_wait(0, 0)
    _g_issue(0, 0)
    _i_issue(1, 1)
    _i_issue(2, 2)

    def _steady(g, carry):
        for u in range(6):
            _step(6 * g + u, u % 2, u % 3, True, True)
        return carry

    NSTEADY = (NCHUNK - 5) // 6
    lax.fori_loop(0, NSTEADY, _steady, 0)
    for c in range(6 * NSTEADY, NCHUNK):
        _step(c, c % 2, c % 3, c + 1 < NCHUNK, c + 3 < NCHUNK)
    plsc.subcore_barrier()

    for r in range(NRCH):
        row0 = sid * RPT + r * RCH
        pltpu.async_copy(agg_sh.at[pl.ds(row0, RCH)],
                         out_hbm.at[cid, pl.ds(row0, RCH)], osem)
    for r in range(NRCH):
        row0 = sid * RPT + r * RCH
        pltpu.make_async_copy(agg_sh.at[pl.ds(row0, RCH)],
                              out_hbm.at[cid, pl.ds(row0, RCH)], osem).wait()


_sc_scatter = functools.partial(
    pl.kernel,
    out_type=jax.ShapeDtypeStruct((NC, NP, D), jnp.float32),
    mesh=plsc.VectorSubcoreMesh(
        core_axis_name="c", subcore_axis_name="s", num_cores=NC, num_subcores=NS
    ),
    scratch_types=[
        pltpu.VMEM((3, K), jnp.int32),
        pltpu.VMEM((3, K), jnp.int32),
        pltpu.VMEM((2, K, D), jnp.float32),
        pltpu.VMEM((RCH, D), jnp.float32),
        pltpu.VMEM_SHARED((NP, D), jnp.float32),
        [pltpu.SemaphoreType.DMA] * 2,
        [pltpu.SemaphoreType.DMA] * 3,
        pltpu.SemaphoreType.DMA,
    ],
)(_sc_body)


BLK = 1000
NBLK = N // BLK


def _tc_body(x_ref, p_ref, b_ref, W1_ref, b1_ref, W2_ref, b2_ref, Wh_ref,
             bh_ref, nl_ref, gl_ref):
    i = pl.program_id(0)
    h0 = x_ref[...] + p_ref[0] + p_ref[1]
    h1 = jnp.dot(h0, W1_ref[...], preferred_element_type=jnp.float32) + b1_ref[...]
    h1 = jnp.maximum(h1, 0.0)
    h2 = jnp.dot(h1, W2_ref[...], preferred_element_type=jnp.float32) + b2_ref[...]
    nl = jnp.dot(h2, Wh_ref[...], preferred_element_type=jnp.float32) + bh_ref[...]
    nl_ref[...] = nl

    seg = b_ref[0, 0, :]
    gids = lax.broadcasted_iota(jnp.int32, (G, BLK), 0)
    onehot = (gids == seg[None, :]).astype(jnp.float32)
    part = jnp.dot(onehot, nl, preferred_element_type=jnp.float32)

    @pl.when(i == 0)
    def _():
        gl_ref[...] = jnp.zeros_like(gl_ref)

    gl_ref[...] += part


def _tc_mlp(x, parts, batch3, W1, b1r, W2, b2r, Wh, bhr):
    full = lambda shape: pl.BlockSpec(shape, lambda i: tuple(0 for _ in shape))
    return pl.pallas_call(
        _tc_body,
        grid=(NBLK,),
        in_specs=[
            pl.BlockSpec((BLK, D), lambda i: (i, 0)),
            pl.BlockSpec((NC, BLK, D), lambda i: (0, i, 0)),
            pl.BlockSpec((1, 1, BLK), lambda i: (i, 0, 0)),
            full((D, H)),
            full((1, H)),
            full((H, H)),
            full((1, H)),
            full((H, 1)),
            full((1, 1)),
        ],
        out_specs=[
            pl.BlockSpec((BLK, 1), lambda i: (i, 0)),
            pl.BlockSpec((G, 1), lambda i: (0, 0)),
        ],
        out_shape=[
            jax.ShapeDtypeStruct((N, 1), jnp.float32),
            jax.ShapeDtypeStruct((G, 1), jnp.float32),
        ],
    )(x, parts, batch3, W1, b1r, W2, b2r, Wh, bhr)


def kernel(x, edge_index, batch, W1, b1, W2, b2, Wh, bh):
    parts = _sc_scatter(x, edge_index[0], edge_index[1])
    batch3 = batch.reshape(NBLK, 1, BLK)
    nl, gl = _tc_mlp(
        x, parts, batch3, W1, b1.reshape(1, H), W2, b2.reshape(1, H), Wh,
        bh.reshape(1, 1),
    )
    return (gl, nl)


# speedup vs baseline: 12.5070x; 1.0481x over previous
import functools

import jax
import jax.numpy as jnp
from jax import lax
from jax.experimental import pallas as pl
from jax.experimental.pallas import tpu as pltpu
from jax.experimental.pallas import tpu_sc as plsc

N = 10000
E = 320000
D = 128
H = 128
G = 64

NC = 2
NS = 16
NW = NC * NS

EPT = E // NW
K = 80
NCHUNK = EPT // K
NP = 10240
RPT = NP // NS
RCH = 128
NRCH = RPT // RCH


def _sc_body(x_hbm, ei_hbm, out_hbm, src_v, dst_v, rows_v, zbuf,
             agg_sh, gsems, isems, osem):
    cid = lax.axis_index("c")
    sid = lax.axis_index("s")

    def _zfill(i, carry):
        for j in range(8):
            zbuf[i, pl.ds(j * 16, 16)] = jnp.zeros((16,), jnp.float32)
        return carry

    lax.fori_loop(0, RCH, _zfill, 0)

    @pl.when(jnp.logical_and(cid == 0, sid < NS - 1))
    def _seed_x_full():
        pltpu.sync_copy(x_hbm.at[pl.ds(sid * RPT, RPT)],
                        agg_sh.at[pl.ds(sid * RPT, RPT)])

    @pl.when(jnp.logical_and(cid == 0, sid == NS - 1))
    def _seed_x_tail():
        pltpu.sync_copy(x_hbm.at[pl.ds(N - 400, 400)],
                        agg_sh.at[pl.ds(N - 400, 400)])
        pltpu.sync_copy(zbuf, agg_sh.at[pl.ds(N, RCH)])
        pltpu.sync_copy(zbuf.at[pl.ds(0, NP - N - RCH)],
                        agg_sh.at[pl.ds(N + RCH, NP - N - RCH)])

    @pl.when(cid != 0)
    def _seed_zero():
        for r in range(NRCH):
            pltpu.sync_copy(zbuf, agg_sh.at[pl.ds(sid * RPT + r * RCH, RCH)])
    plsc.subcore_barrier()

    ebase = (cid * NS + sid) * EPT

    def _i_issue(c, s):
        pltpu.async_copy(ei_hbm.at[pl.ds(ebase + c * K, K)], src_v.at[s],
                         isems[s])
        pltpu.async_copy(ei_hbm.at[pl.ds(E + ebase + c * K, K)], dst_v.at[s],
                         isems[s])

    def _i_wait(c, s):
        pltpu.make_async_copy(ei_hbm.at[pl.ds(ebase + c * K, K)],
                              src_v.at[s], isems[s]).wait()
        pltpu.make_async_copy(ei_hbm.at[pl.ds(E + ebase + c * K, K)],
                              dst_v.at[s], isems[s]).wait()

    def _g_issue(s, b):
        pltpu.async_copy(x_hbm.at[src_v.at[s]], rows_v.at[b], gsems[b])

    def _g_wait(s, b):
        pltpu.make_async_copy(x_hbm.at[src_v.at[s]], rows_v.at[b],
                              gsems[b]).wait()

    def _scatter(s, b):
        pltpu.sync_copy(rows_v.at[b], agg_sh.at[dst_v.at[s]], add=True)

    def _step(c, b, s, do_next, do_pref):
        if do_next:
            _i_wait(c + 1, (s + 1) % 3)
            _g_issue((s + 1) % 3, 1 - b)
        _g_wait(s, b)
        _scatter(s, b)
        if do_pref:
            _i_issue(c + 3, s)

    _i_issue(0, 0)
    _i_wait(0, 0)
    _g_issue(0, 0)
    _i_issue(1, 1)
    _i_issue(2, 2)

    def _steady(g, carry):
        for u in range(6):
            _step(6 * g + u, u % 2, u % 3, True, True)
        return carry

    NSTEADY = (NCHUNK - 5) // 6
    lax.fori_loop(0, NSTEADY, _steady, 0)
    for c in range(6 * NSTEADY, NCHUNK):
        _step(c, c % 2, c % 3, c + 1 < NCHUNK, c + 3 < NCHUNK)
    plsc.subcore_barrier()

    for r in range(NRCH):
        row0 = sid * RPT + r * RCH
        pltpu.async_copy(agg_sh.at[pl.ds(row0, RCH)],
                         out_hbm.at[cid, pl.ds(row0, RCH)], osem)
    for r in range(NRCH):
        row0 = sid * RPT + r * RCH
        pltpu.make_async_copy(agg_sh.at[pl.ds(row0, RCH)],
                              out_hbm.at[cid, pl.ds(row0, RCH)], osem).wait()


_sc_scatter = functools.partial(
    pl.kernel,
    out_type=jax.ShapeDtypeStruct((NC, NP, D), jnp.float32),
    mesh=plsc.VectorSubcoreMesh(
        core_axis_name="c", subcore_axis_name="s", num_cores=NC, num_subcores=NS
    ),
    scratch_types=[
        pltpu.VMEM((3, K), jnp.int32),
        pltpu.VMEM((3, K), jnp.int32),
        pltpu.VMEM((2, K, D), jnp.float32),
        pltpu.VMEM((RCH, D), jnp.float32),
        pltpu.VMEM_SHARED((NP, D), jnp.float32),
        [pltpu.SemaphoreType.DMA] * 2,
        [pltpu.SemaphoreType.DMA] * 3,
        pltpu.SemaphoreType.DMA,
    ],
)(_sc_body)


BLK = 1000
NBLK = N // BLK


def _tc_body(p_ref, b_ref, W1_ref, b1_ref, W2_ref, b2_ref, Wh_ref,
             bh_ref, nl_ref, gl_ref):
    i = pl.program_id(0)
    h0 = p_ref[0] + p_ref[1]
    h1 = jnp.dot(h0, W1_ref[...], preferred_element_type=jnp.float32) + b1_ref[...]
    h1 = jnp.maximum(h1, 0.0)
    h2 = jnp.dot(h1, W2_ref[...], preferred_element_type=jnp.float32) + b2_ref[...]
    nl = jnp.dot(h2, Wh_ref[...], preferred_element_type=jnp.float32) + bh_ref[...]
    nl_ref[...] = nl

    seg = b_ref[0, 0, :]
    gids = lax.broadcasted_iota(jnp.int32, (G, BLK), 0)
    onehot = (gids == seg[None, :]).astype(jnp.float32)
    part = jnp.dot(onehot, nl, preferred_element_type=jnp.float32)

    @pl.when(i == 0)
    def _():
        gl_ref[...] = jnp.zeros_like(gl_ref)

    gl_ref[...] += part


def _tc_mlp(parts, batch3, W1, b1r, W2, b2r, Wh, bhr):
    full = lambda shape: pl.BlockSpec(shape, lambda i: tuple(0 for _ in shape))
    return pl.pallas_call(
        _tc_body,
        grid=(NBLK,),
        in_specs=[
            pl.BlockSpec((NC, BLK, D), lambda i: (0, i, 0)),
            pl.BlockSpec((1, 1, BLK), lambda i: (i, 0, 0)),
            full((D, H)),
            full((1, H)),
            full((H, H)),
            full((1, H)),
            full((H, 1)),
            full((1, 1)),
        ],
        out_specs=[
            pl.BlockSpec((BLK, 1), lambda i: (i, 0)),
            pl.BlockSpec((G, 1), lambda i: (0, 0)),
        ],
        out_shape=[
            jax.ShapeDtypeStruct((N, 1), jnp.float32),
            jax.ShapeDtypeStruct((G, 1), jnp.float32),
        ],
    )(parts, batch3, W1, b1r, W2, b2r, Wh, bhr)


def kernel(x, edge_index, batch, W1, b1, W2, b2, Wh, bh):
    parts = _sc_scatter(x, edge_index.reshape(2 * E))
    batch3 = batch.reshape(NBLK, 1, BLK)
    nl, gl = _tc_mlp(
        parts, batch3, W1, b1.reshape(1, H), W2, b2.reshape(1, H), Wh,
        bh.reshape(1, 1),
    )
    return (gl, nl)


# speedup vs baseline: 13.1255x; 1.0495x over previous
import functools

import jax
import jax.numpy as jnp
from jax import lax
from jax.experimental import pallas as pl
from jax.experimental.pallas import tpu as pltpu
from jax.experimental.pallas import tpu_sc as plsc

N = 10000
E = 320000
D = 128
H = 128
G = 64

NC = 2
NS = 16
NW = NC * NS

EPT = E // NW
K = 80
NCHUNK = EPT // K
NP = 10240
RPT = NP // NS
RCH = 128
NRCH = RPT // RCH


def _sc_body(x_hbm, ei_hbm, out_hbm, src_v, dst_v, rows_v, zbuf,
             agg_sh, gsems, isems, osem):
    cid = lax.axis_index("c")
    sid = lax.axis_index("s")

    def _zfill(i, carry):
        for j in range(8):
            zbuf[i, pl.ds(j * 16, 16)] = jnp.zeros((16,), jnp.float32)
        return carry

    ebase = (cid * NS + sid) * EPT

    def _i_issue(c, s):
        pltpu.async_copy(ei_hbm.at[pl.ds(ebase + c * K, K)], src_v.at[s],
                         isems[s])
        pltpu.async_copy(ei_hbm.at[pl.ds(E + ebase + c * K, K)], dst_v.at[s],
                         isems[s])

    def _i_wait(c, s):
        pltpu.make_async_copy(ei_hbm.at[pl.ds(ebase + c * K, K)],
                              src_v.at[s], isems[s]).wait()
        pltpu.make_async_copy(ei_hbm.at[pl.ds(E + ebase + c * K, K)],
                              dst_v.at[s], isems[s]).wait()

    def _g_issue(s, b):
        pltpu.async_copy(x_hbm.at[src_v.at[s]], rows_v.at[b], gsems[b])

    def _g_wait(s, b):
        pltpu.make_async_copy(x_hbm.at[src_v.at[s]], rows_v.at[b],
                              gsems[b]).wait()

    def _scatter(s, b):
        pltpu.sync_copy(rows_v.at[b], agg_sh.at[dst_v.at[s]], add=True)

    def _step(c, b, s, do_next, do_pref):
        if do_next:
            _i_wait(c + 1, (s + 1) % 3)
            _g_issue((s + 1) % 3, 1 - b)
        _g_wait(s, b)
        _scatter(s, b)
        if do_pref:
            _i_issue(c + 3, s)

    _i_issue(0, 0)
    _i_issue(1, 1)
    _i_issue(2, 2)
    lax.fori_loop(0, RCH, _zfill, 0)
    _i_wait(0, 0)
    _g_issue(0, 0)

    seed_mine = jnp.where(cid == 0, sid < NS // 2,
                          jnp.logical_and(sid >= NS // 2, sid < NS - 1))

    @pl.when(seed_mine)
    def _seed_x_full():
        pltpu.sync_copy(x_hbm.at[pl.ds(sid * RPT, RPT)],
                        agg_sh.at[pl.ds(sid * RPT, RPT)])

    @pl.when(jnp.logical_and(cid == 1, sid == NS - 1))
    def _seed_x_tail():
        pltpu.sync_copy(x_hbm.at[pl.ds(N - 400, 400)],
                        agg_sh.at[pl.ds(N - 400, 400)])
        pltpu.sync_copy(zbuf, agg_sh.at[pl.ds(N, RCH)])
        pltpu.sync_copy(zbuf.at[pl.ds(0, NP - N - RCH)],
                        agg_sh.at[pl.ds(N + RCH, NP - N - RCH)])

    @pl.when(jnp.logical_not(jnp.logical_or(
        seed_mine, jnp.logical_and(cid == 1, sid == NS - 1))))
    def _seed_zero():
        for r in range(NRCH):
            pltpu.sync_copy(zbuf, agg_sh.at[pl.ds(sid * RPT + r * RCH, RCH)])
    plsc.subcore_barrier()

    def _steady(g, carry):
        for u in range(6):
            _step(6 * g + u, u % 2, u % 3, True, True)
        return carry

    NSTEADY = (NCHUNK - 5) // 6
    lax.fori_loop(0, NSTEADY, _steady, 0)
    for c in range(6 * NSTEADY, NCHUNK):
        _step(c, c % 2, c % 3, c + 1 < NCHUNK, c + 3 < NCHUNK)
    plsc.subcore_barrier()

    for r in range(NRCH):
        row0 = sid * RPT + r * RCH
        pltpu.async_copy(agg_sh.at[pl.ds(row0, RCH)],
                         out_hbm.at[cid, pl.ds(row0, RCH)], osem)
    for r in range(NRCH):
        row0 = sid * RPT + r * RCH
        pltpu.make_async_copy(agg_sh.at[pl.ds(row0, RCH)],
                              out_hbm.at[cid, pl.ds(row0, RCH)], osem).wait()


_sc_scatter = functools.partial(
    pl.kernel,
    out_type=jax.ShapeDtypeStruct((NC, NP, D), jnp.float32),
    mesh=plsc.VectorSubcoreMesh(
        core_axis_name="c", subcore_axis_name="s", num_cores=NC, num_subcores=NS
    ),
    scratch_types=[
        pltpu.VMEM((3, K), jnp.int32),
        pltpu.VMEM((3, K), jnp.int32),
        pltpu.VMEM((2, K, D), jnp.float32),
        pltpu.VMEM((RCH, D), jnp.float32),
        pltpu.VMEM_SHARED((NP, D), jnp.float32),
        [pltpu.SemaphoreType.DMA] * 2,
        [pltpu.SemaphoreType.DMA] * 3,
        pltpu.SemaphoreType.DMA,
    ],
)(_sc_body)


BLK = 2000
NBLK = N // BLK


def _tc_body(p_ref, b_ref, W1_ref, b1_ref, W2_ref, b2_ref, Wh_ref,
             bh_ref, nl_ref, gl_ref):
    i = pl.program_id(0)
    h0 = p_ref[0] + p_ref[1]
    h1 = jnp.dot(h0, W1_ref[...], preferred_element_type=jnp.float32) + b1_ref[...]
    h1 = jnp.maximum(h1, 0.0)
    h2 = jnp.dot(h1, W2_ref[...], preferred_element_type=jnp.float32) + b2_ref[...]
    nl = jnp.dot(h2, Wh_ref[...], preferred_element_type=jnp.float32) + bh_ref[...]
    nl_ref[...] = nl

    seg = b_ref[0, 0, :]
    gids = lax.broadcasted_iota(jnp.int32, (G, BLK), 0)
    onehot = (gids == seg[None, :]).astype(jnp.float32)
    part = jnp.dot(onehot, nl, preferred_element_type=jnp.float32)

    @pl.when(i == 0)
    def _():
        gl_ref[...] = jnp.zeros_like(gl_ref)

    gl_ref[...] += part


def _tc_mlp(parts, batch3, W1, b1r, W2, b2r, Wh, bhr):
    full = lambda shape: pl.BlockSpec(shape, lambda i: tuple(0 for _ in shape))
    return pl.pallas_call(
        _tc_body,
        grid=(NBLK,),
        in_specs=[
            pl.BlockSpec((NC, BLK, D), lambda i: (0, i, 0)),
            pl.BlockSpec((1, 1, BLK), lambda i: (i, 0, 0)),
            full((D, H)),
            full((1, H)),
            full((H, H)),
            full((1, H)),
            full((H, 1)),
            full((1, 1)),
        ],
        out_specs=[
            pl.BlockSpec((BLK, 1), lambda i: (i, 0)),
            pl.BlockSpec((G, 1), lambda i: (0, 0)),
        ],
        out_shape=[
            jax.ShapeDtypeStruct((N, 1), jnp.float32),
            jax.ShapeDtypeStruct((G, 1), jnp.float32),
        ],
    )(parts, batch3, W1, b1r, W2, b2r, Wh, bhr)


def kernel(x, edge_index, batch, W1, b1, W2, b2, Wh, bh):
    parts = _sc_scatter(x, edge_index.reshape(2 * E))
    batch3 = batch.reshape(NBLK, 1, BLK)
    nl, gl = _tc_mlp(
        parts, batch3, W1, b1.reshape(1, H), W2, b2.reshape(1, H), Wh,
        bh.reshape(1, 1),
    )
    return (gl, nl)


# speedup vs baseline: 15.4081x; 1.1739x over previous
import functools

import jax
import jax.numpy as jnp
from jax import lax
from jax.experimental import pallas as pl
from jax.experimental.pallas import tpu as pltpu
from jax.experimental.pallas import tpu_sc as plsc

N = 10000
E = 320000
D = 128
H = 128
G = 64

NC = 2
NS = 16
NW = NC * NS

EPT = E // NW
K = 80
NCHUNK = EPT // K
NP = 10240
RPT = NP // NS
RCH = 128
NRCH = RPT // RCH


def _sc_body(x_hbm, ei_hbm, out_hbm, src_v, dst_v, rows_v, zbuf,
             agg_sh, gsems, isems, osem):
    cid = lax.axis_index("c")
    sid = lax.axis_index("s")

    def _zfill(i, carry):
        for j in range(8):
            zbuf[i, pl.ds(j * 16, 16)] = jnp.zeros((16,), jnp.float32)
        return carry

    ebase = (cid * NS + sid) * EPT

    def _i_issue(c, s):
        pltpu.async_copy(ei_hbm.at[pl.ds(ebase + c * K, K)], src_v.at[s],
                         isems[s])
        pltpu.async_copy(ei_hbm.at[pl.ds(E + ebase + c * K, K)], dst_v.at[s],
                         isems[s])

    def _i_wait(c, s):
        pltpu.make_async_copy(ei_hbm.at[pl.ds(ebase + c * K, K)],
                              src_v.at[s], isems[s]).wait()
        pltpu.make_async_copy(ei_hbm.at[pl.ds(E + ebase + c * K, K)],
                              dst_v.at[s], isems[s]).wait()

    def _g_issue(s, b):
        pltpu.async_copy(x_hbm.at[src_v.at[s]], rows_v.at[b], gsems[b])

    def _g_wait(s, b):
        pltpu.make_async_copy(x_hbm.at[src_v.at[s]], rows_v.at[b],
                              gsems[b]).wait()

    def _scatter(s, b):
        pltpu.sync_copy(rows_v.at[b], agg_sh.at[dst_v.at[s]], add=True)

    def _step(c, b, s, do_next, do_pref):
        if do_next:
            _i_wait(c + 2, (s + 2) % 4)
            _g_issue((s + 2) % 4, (b + 2) % 3)
        _g_wait(s, b)
        _scatter(s, b)
        if do_pref:
            _i_issue(c + 4, s)

    _i_issue(0, 0)
    _i_issue(1, 1)
    _i_issue(2, 2)
    _i_issue(3, 3)
    lax.fori_loop(0, RCH, _zfill, 0)
    _i_wait(0, 0)
    _g_issue(0, 0)
    _i_wait(1, 1)
    _g_issue(1, 1)

    seed_mine = jnp.where(cid == 0, sid < NS // 2,
                          jnp.logical_and(sid >= NS // 2, sid < NS - 1))

    @pl.when(seed_mine)
    def _seed_x_full():
        pltpu.sync_copy(x_hbm.at[pl.ds(sid * RPT, RPT)],
                        agg_sh.at[pl.ds(sid * RPT, RPT)])

    @pl.when(jnp.logical_and(cid == 1, sid == NS - 1))
    def _seed_x_tail():
        pltpu.sync_copy(x_hbm.at[pl.ds(N - 400, 400)],
                        agg_sh.at[pl.ds(N - 400, 400)])
        pltpu.sync_copy(zbuf, agg_sh.at[pl.ds(N, RCH)])
        pltpu.sync_copy(zbuf.at[pl.ds(0, NP - N - RCH)],
                        agg_sh.at[pl.ds(N + RCH, NP - N - RCH)])

    @pl.when(jnp.logical_not(jnp.logical_or(
        seed_mine, jnp.logical_and(cid == 1, sid == NS - 1))))
    def _seed_zero():
        for r in range(NRCH):
            pltpu.sync_copy(zbuf, agg_sh.at[pl.ds(sid * RPT + r * RCH, RCH)])
    plsc.subcore_barrier()

    def _steady(g, carry):
        for u in range(12):
            _step(12 * g + u, u % 3, u % 4, True, True)
        return carry

    NSTEADY = (NCHUNK - 5) // 12
    lax.fori_loop(0, NSTEADY, _steady, 0)
    for c in range(12 * NSTEADY, NCHUNK):
        _step(c, c % 3, c % 4, c + 2 < NCHUNK, c + 4 < NCHUNK)
    plsc.subcore_barrier()

    for r in range(NRCH):
        row0 = sid * RPT + r * RCH
        pltpu.async_copy(agg_sh.at[pl.ds(row0, RCH)],
                         out_hbm.at[cid, pl.ds(row0, RCH)], osem)
    for r in range(NRCH):
        row0 = sid * RPT + r * RCH
        pltpu.make_async_copy(agg_sh.at[pl.ds(row0, RCH)],
                              out_hbm.at[cid, pl.ds(row0, RCH)], osem).wait()


_sc_scatter = functools.partial(
    pl.kernel,
    out_type=jax.ShapeDtypeStruct((NC, NP, D), jnp.float32),
    mesh=plsc.VectorSubcoreMesh(
        core_axis_name="c", subcore_axis_name="s", num_cores=NC, num_subcores=NS
    ),
    scratch_types=[
        pltpu.VMEM((4, K), jnp.int32),
        pltpu.VMEM((4, K), jnp.int32),
        pltpu.VMEM((3, K, D), jnp.float32),
        pltpu.VMEM((RCH, D), jnp.float32),
        pltpu.VMEM_SHARED((NP, D), jnp.float32),
        [pltpu.SemaphoreType.DMA] * 3,
        [pltpu.SemaphoreType.DMA] * 4,
        pltpu.SemaphoreType.DMA,
    ],
)(_sc_body)


BLK = 2000
NBLK = N // BLK


def _tc_body(p_ref, b_ref, W1_ref, b1_ref, W2_ref, b2_ref, Wh_ref,
             bh_ref, nl_ref, gl_ref):
    i = pl.program_id(0)
    h0 = p_ref[0] + p_ref[1]
    h1 = jnp.dot(h0, W1_ref[...], preferred_element_type=jnp.float32) + b1_ref[...]
    h1 = jnp.maximum(h1, 0.0)
    h2 = jnp.dot(h1, W2_ref[...], preferred_element_type=jnp.float32) + b2_ref[...]
    nl = jnp.dot(h2, Wh_ref[...], preferred_element_type=jnp.float32) + bh_ref[...]
    nl_ref[...] = nl

    seg = b_ref[0, 0, :]
    gids = lax.broadcasted_iota(jnp.int32, (G, BLK), 0)
    onehot = (gids == seg[None, :]).astype(jnp.float32)
    part = jnp.dot(onehot, nl, preferred_element_type=jnp.float32)

    @pl.when(i == 0)
    def _():
        gl_ref[...] = jnp.zeros_like(gl_ref)

    gl_ref[...] += part


def _tc_mlp(parts, batch3, W1, b1r, W2, b2r, Wh, bhr):
    full = lambda shape: pl.BlockSpec(shape, lambda i: tuple(0 for _ in shape))
    return pl.pallas_call(
        _tc_body,
        grid=(NBLK,),
        in_specs=[
            pl.BlockSpec((NC, BLK, D), lambda i: (0, i, 0)),
            pl.BlockSpec((1, 1, BLK), lambda i: (i, 0, 0)),
            full((D, H)),
            full((1, H)),
            full((H, H)),
            full((1, H)),
            full((H, 1)),
            full((1, 1)),
        ],
        out_specs=[
            pl.BlockSpec((BLK, 1), lambda i: (i, 0)),
            pl.BlockSpec((G, 1), lambda i: (0, 0)),
        ],
        out_shape=[
            jax.ShapeDtypeStruct((N, 1), jnp.float32),
            jax.ShapeDtypeStruct((G, 1), jnp.float32),
        ],
    )(parts, batch3, W1, b1r, W2, b2r, Wh, bhr)


def kernel(x, edge_index, batch, W1, b1, W2, b2, Wh, bh):
    parts = _sc_scatter(x, edge_index.reshape(2 * E))
    batch3 = batch.reshape(NBLK, 1, BLK)
    nl, gl = _tc_mlp(
        parts, batch3, W1, b1.reshape(1, H), W2, b2.reshape(1, H), Wh,
        bh.reshape(1, 1),
    )
    return (gl, nl)
